# baseline (speedup 1.0000x reference)
import functools

import jax
import jax.numpy as jnp
from jax import lax
from jax.experimental import pallas as pl
from jax.experimental.pallas import tpu as pltpu
from jax.experimental.pallas import tpu_sc as plsc

_N = 1000000
_D = 16
_B = 16384
_NS = 16
_PER_TILE = _B // _NS
_CH = 128
_NCH = _PER_TILE // _CH
_ROWS_PER_TILE = _PER_TILE // _CH
_TCC = 8192

_mesh = plsc.VectorSubcoreMesh(core_axis_name="c", subcore_axis_name="s")


def _detile(xt, n):
    grid = -(-n // _TCC)

    def body(x_ref, *o_refs):
        x = x_ref[...]
        for d in range(_D):
            o_refs[d][...] = x[d, :]

    return pl.pallas_call(
        body,
        grid=(grid,),
        in_specs=[pl.BlockSpec((_D, _TCC), lambda i: (0, i))],
        out_specs=[pl.BlockSpec((_TCC,), lambda i: (i,)) for _ in range(_D)],
        out_shape=[jax.ShapeDtypeStruct((n,), jnp.float32) for _ in range(_D)],
    )(xt)


def _retile(parts, n):
    grid = -(-n // _TCC)

    def body(*refs):
        o_ref = refs[-1]
        for d in range(_D):
            o_ref[d, :] = refs[d][...]

    return pl.pallas_call(
        body,
        grid=(grid,),
        in_specs=[pl.BlockSpec((_TCC,), lambda i: (i,)) for _ in range(_D)],
        out_specs=pl.BlockSpec((_D, _TCC), lambda i: (0, i)),
        out_shape=jax.ShapeDtypeStruct((_D, n), jnp.float32),
    )(*parts)


@functools.partial(
    pl.kernel,
    out_type=[jax.ShapeDtypeStruct((_B,), jnp.float32) for _ in range(_D)],
    mesh=_mesh,
    scratch_types=[
        pltpu.VMEM((_NCH // 2, _CH), jnp.int32),
        pltpu.VMEM((_D, _PER_TILE // 2), jnp.float32),
        pltpu.SemaphoreType.DMA,
    ],
    compiler_params=pltpu.CompilerParams(use_tc_tiling_on_sc=False, needs_layout_passes=False),
)
def _sc_gather(node2d, *args):
    stages = args[:_D]
    embs = args[_D:2 * _D]
    idx_v, val_v, sem = args[2 * _D:]
    c = lax.axis_index("c")
    s = lax.axis_index("s")
    w = s * 2 + c
    per_w = _PER_TILE // 2
    nch = _NCH // 2
    base = w * per_w
    rowbase = w * (per_w // _CH)

    pltpu.sync_copy(node2d.at[pl.ds(rowbase, nch)], idx_v)
    for j in range(nch):
        descs = [
            pltpu.async_copy(
                stages[d].at[idx_v.at[j]],
                val_v.at[d, pl.ds(j * _CH, _CH)],
                sem,
            )
            for d in range(_D)
        ]
        for desc in descs:
            desc.wait()
    for d in range(_D):
        pltpu.sync_copy(val_v.at[d], embs[d].at[pl.ds(base, per_w)])


@functools.partial(
    pl.kernel,
    out_type=(),
    mesh=_mesh,
    scratch_types=[
        pltpu.VMEM_SHARED((_N,), jnp.int32),
        pltpu.VMEM_SHARED((_D // 2, _B), jnp.float32),
        pltpu.VMEM_SHARED((_B,), jnp.float32),
        pltpu.VMEM((_NCH, _CH), jnp.int32),
        pltpu.VMEM((_NCH, _CH), jnp.int32),
        pltpu.VMEM((_NCH, _CH), jnp.int32),
        pltpu.VMEM((_D // 2, _PER_TILE), jnp.float32),
        pltpu.VMEM((_D // 2, _PER_TILE), jnp.float32),
        pltpu.VMEM((_PER_TILE,), jnp.float32),
        pltpu.VMEM((_PER_TILE,), jnp.float32),
        pltpu.VMEM((_PER_TILE,), jnp.float32),
        pltpu.VMEM((_CH,), jnp.float32),
        pltpu.SemaphoreType.DMA,
    ],
    compiler_params=pltpu.CompilerParams(use_tc_tiling_on_sc=False, needs_layout_passes=False),
)
def _sc_update(node2d, *args):
    ustages = args[:_D]
    outw = args[_D:2 * _D]
    (slot_tab, acc, cnt, idx_v, ids_v, rep_v, upd_v, val_v, cnt_v, rcp_v,
     zrow_v, oc_v, sem) = args[2 * _D:]
    c = lax.axis_index("c")
    s = lax.axis_index("s")
    base = s * _PER_TILE
    rowbase = s * _ROWS_PER_TILE
    ngrp = _PER_TILE // _D
    nf = _D // 2
    lanes = lax.iota(jnp.int32, _D)

    def phase_a(fb):
        with jax.named_scope("ph_a_idx"):
            pltpu.sync_copy(node2d.at[pl.ds(rowbase, _NCH)], idx_v)
        with jax.named_scope("ph_a_fill"):
            for j in range(_NCH):
                for g in range(_CH // _D):
                    ids_v[j, pl.ds(g * _D, _D)] = (base + j * _CH + g * _D
                                                   + lanes)
            for g in range(_CH // _D):
                oc_v[pl.ds(g * _D, _D)] = jnp.ones((_D,), jnp.float32)
            for g in range(ngrp):
                zrow_v[pl.ds(g * _D, _D)] = jnp.zeros((_D,), jnp.float32)
        with jax.named_scope("ph_a_upd"):
            for d in range(nf):
                pltpu.sync_copy(ustages[fb + d].at[pl.ds(base, _PER_TILE)],
                                upd_v.at[d])
        with jax.named_scope("ph_a_zero"):
            for d in range(nf):
                pltpu.sync_copy(zrow_v, acc.at[d, pl.ds(base, _PER_TILE)])
            pltpu.sync_copy(zrow_v, cnt.at[pl.ds(base, _PER_TILE)])
        with jax.named_scope("ph_a_slot"):
            for j in range(_NCH):
                pltpu.sync_copy(ids_v.at[j], slot_tab.at[idx_v.at[j]])

    def phase_b(fb):
        with jax.named_scope("ph_b_rep"):
            for j in range(_NCH):
                pltpu.sync_copy(slot_tab.at[idx_v.at[j]], rep_v.at[j])
        with jax.named_scope("ph_b_add"):
            for j in range(_NCH):
                descs = [
                    pltpu.async_copy(upd_v.at[d, pl.ds(j * _CH, _CH)],
                                     acc.at[d].at[rep_v.at[j]], sem, add=True)
                    for d in range(nf)
                ]
                descs.append(pltpu.async_copy(oc_v, cnt.at[rep_v.at[j]], sem,
                                              add=True))
                for desc in descs:
                    desc.wait()

    def phase_c(fb):
        with jax.named_scope("ph_c_gather"):
            for j in range(_NCH):
                descs = [
                    pltpu.async_copy(acc.at[d].at[rep_v.at[j]],
                                     val_v.at[d, pl.ds(j * _CH, _CH)], sem)
                    for d in range(nf)
                ]
                descs.append(
                    pltpu.async_copy(cnt.at[rep_v.at[j]],
                                     cnt_v.at[pl.ds(j * _CH, _CH)], sem))
                for desc in descs:
                    desc.wait()

        def _recip(g, _):
            gs = pl.ds(g * _D, _D)
            rcp_v[gs] = 1.0 / cnt_v[gs]
            return 0

        with jax.named_scope("ph_c_math"):
            lax.fori_loop(0, ngrp, _recip, 0)

        def _scale(g, _):
            gs = pl.ds(g * _D, _D)
            r = rcp_v[gs]
            for d in range(nf):
                val_v[d, gs] = val_v[d, gs] * r
            return 0

        with jax.named_scope("ph_c_math2"):
            lax.fori_loop(0, ngrp, _scale, 0)
        with jax.named_scope("ph_c_scatter"):
            for j in range(_NCH):
                descs = [
                    pltpu.async_copy(val_v.at[d, pl.ds(j * _CH, _CH)],
                                     outw[fb + d].at[idx_v.at[j]], sem)
                    for d in range(nf)
                ]
                for desc in descs:
                    desc.wait()

    @pl.when(c == 0)
    def _():
        phase_a(0)

    @pl.when(c == 1)
    def _():
        phase_a(nf)

    plsc.subcore_barrier()

    @pl.when(c == 0)
    def _():
        phase_b(0)

    @pl.when(c == 1)
    def _():
        phase_b(nf)

    plsc.subcore_barrier()

    @pl.when(c == 0)
    def _():
        phase_c(0)

    @pl.when(c == 1)
    def _():
        phase_c(nf)


def kernel(weight, node, update):
    wt = weight.T
    updt = update.T
    node2d = node.reshape(_B // _CH, _CH)
    stages = _detile(wt, _N)
    ustages = _detile(updt, _B)
    outw = [jax.new_ref(st) for st in stages]
    embs = _sc_gather(node2d, *stages)
    _sc_update(node2d, *ustages, *outw)
    embt = _retile(embs, _B)
    new_wt = _retile([jax.freeze(r) for r in outw], _N)
    return embt.T, new_wt.T
